# Initial kernel scaffold; baseline (speedup 1.0000x reference)
import jax
import jax.numpy as jnp
from jax.experimental import pallas as pl


def kernel(x, edge_index, etypes, weight, w_comp, loop_weight, h_bias):
    raise NotImplementedError("write your pallas kernel here")


# speedup vs baseline: 6.5115x; 6.5115x over previous
import functools

import jax
import jax.numpy as jnp
from jax import lax
from jax.experimental import pallas as pl
from jax.experimental.pallas import tpu as pltpu
from jax.experimental.pallas import tpu_sc as plsc

N = 10000
E = 320000
D = 128
R = 8
NB = 4

NC = 2
NS = 16
NW = NC * NS

K = 80
EPW = E // NW
CPT = EPW // K
RPT = N // NS
SP = 25


def _wcomb_body(wc_ref, w_ref, wall_ref):
    w = w_ref[...]
    for r in range(R):
        acc = wc_ref[r, 0] * w[0]
        for b in range(1, NB):
            acc = acc + wc_ref[r, b] * w[b]
        wall_ref[r] = acc


def _wcomb(w_comp, weight):
    return pl.pallas_call(
        _wcomb_body,
        out_shape=jax.ShapeDtypeStruct((R, D, D), jnp.float32),
        in_specs=[
            pl.BlockSpec(memory_space=pltpu.SMEM),
            pl.BlockSpec(memory_space=pltpu.VMEM),
        ],
        out_specs=pl.BlockSpec(memory_space=pltpu.VMEM),
    )(w_comp, weight)


BN = 400


def _prep_body(x_ref, wall_ref, lw_ref, bias_ref, y2_ref, hloop_ref):
    xb = x_ref[...]
    for r in range(R):
        y2_ref[:, r, :] = jnp.dot(xb, wall_ref[r], preferred_element_type=jnp.float32)
    hloop_ref[...] = (
        jnp.dot(xb, lw_ref[...], preferred_element_type=jnp.float32) + bias_ref[...]
    )


def _prep(x, wall, loop_weight, h_bias2d):
    grid = N // BN
    return pl.pallas_call(
        _prep_body,
        grid=(grid,),
        in_specs=[
            pl.BlockSpec((BN, D), lambda i: (i, 0)),
            pl.BlockSpec((R, D, D), lambda i: (0, 0, 0)),
            pl.BlockSpec((D, D), lambda i: (0, 0)),
            pl.BlockSpec((1, D), lambda i: (0, 0)),
        ],
        out_specs=[
            pl.BlockSpec((BN, R, D), lambda i: (i, 0, 0)),
            pl.BlockSpec((BN, D), lambda i: (i, 0)),
        ],
        out_shape=[
            jax.ShapeDtypeStruct((N, R, D), jnp.float32),
            jax.ShapeDtypeStruct((N, D), jnp.float32),
        ],
    )(x, wall, loop_weight, h_bias2d)


def _gidx_body(src_ref, et_ref, gidx_ref):
    gidx_ref[...] = src_ref[...] * R + et_ref[...]


def _gidx(src2d, et2d):
    return pl.pallas_call(
        _gidx_body,
        out_shape=jax.ShapeDtypeStruct(src2d.shape, jnp.int32),
    )(src2d, et2d)


def _sc_body(y2_hbm, gidx_hbm, dst_hbm, zeros_hbm, out_hbm,
             gidx_v, dst_v, rows_v, stage_v, h_sh, sem):
    cid = lax.axis_index("c")
    sid = lax.axis_index("s")
    wid = sid * NC + cid

    pltpu.sync_copy(gidx_hbm.at[wid], gidx_v)
    pltpu.sync_copy(dst_hbm.at[wid], dst_v)

    rbase = sid * RPT
    pltpu.sync_copy(zeros_hbm, stage_v)
    for p in range(RPT // SP):
        pltpu.sync_copy(stage_v, h_sh.at[pl.ds(rbase + p * SP, SP)])
    plsc.subcore_barrier()

    def chunk(j, carry):
        pltpu.async_copy(y2_hbm.at[gidx_v.at[j]], rows_v, sem).wait()
        pltpu.sync_copy(rows_v, h_sh.at[dst_v.at[j]], add=True)
        return carry

    lax.fori_loop(0, CPT, chunk, 0)
    plsc.subcore_barrier()

    for p in range(RPT // SP):
        pltpu.sync_copy(h_sh.at[pl.ds(rbase + p * SP, SP)], stage_v)
        pltpu.sync_copy(stage_v, out_hbm.at[cid, sid, p])


@functools.cache
def _sc_scatter_kernel():
    return pl.kernel(
        _sc_body,
        out_type=jax.ShapeDtypeStruct((NC, NS, RPT // SP, SP, D), jnp.float32),
        mesh=plsc.VectorSubcoreMesh(
            core_axis_name="c", subcore_axis_name="s", num_cores=NC, num_subcores=NS
        ),
        scratch_types=[
            pltpu.VMEM((CPT, K), jnp.int32),
            pltpu.VMEM((CPT, K), jnp.int32),
            pltpu.VMEM((K, D), jnp.float32),
            pltpu.VMEM((SP, D), jnp.float32),
            pltpu.VMEM_SHARED((N, D), jnp.float32),
            pltpu.SemaphoreType.DMA,
        ],
    )


FBN = 2000


def _final_body(p_ref, hl_ref, out_ref):
    out_ref[...] = p_ref[0] + p_ref[1] + hl_ref[...]


def _final(partial, h_loop):
    return pl.pallas_call(
        _final_body,
        grid=(N // FBN,),
        in_specs=[
            pl.BlockSpec((NC, FBN, D), lambda i: (0, i, 0)),
            pl.BlockSpec((FBN, D), lambda i: (i, 0)),
        ],
        out_specs=pl.BlockSpec((FBN, D), lambda i: (i, 0)),
        out_shape=jax.ShapeDtypeStruct((N, D), jnp.float32),
    )(partial, h_loop)


def kernel(x, edge_index, etypes, weight, w_comp, loop_weight, h_bias):
    wall = _wcomb(w_comp, weight)
    y2, h_loop = _prep(x, wall, loop_weight, h_bias.reshape(1, D))
    gidx = _gidx(edge_index[0].reshape(E // D, D), etypes.reshape(E // D, D))
    partial = _sc_scatter_kernel()(
        y2.reshape(N * R, D),
        gidx.reshape(NW, CPT, K),
        edge_index[1].reshape(NW, CPT, K),
        jnp.zeros((SP, D), jnp.float32),
    )
    return _final(partial.reshape(NC, N, D), h_loop)


# speedup vs baseline: 10.3159x; 1.5843x over previous
import functools

import jax
import jax.numpy as jnp
from jax import lax
from jax.experimental import pallas as pl
from jax.experimental.pallas import tpu as pltpu
from jax.experimental.pallas import tpu_sc as plsc

N = 10000
E = 320000
D = 128
R = 8
NB = 4

NC = 2
NS = 16
NW = NC * NS

K = 100
EPW = E // NW
PH = 2
CPP = EPW // (PH * K)
ZB = 640
ZL = N - (NS - 1) * ZB


def _wcomb_body(wc_ref, w_ref, wall_ref):
    w = w_ref[...]
    for r in range(R):
        acc = wc_ref[r, 0] * w[0]
        for b in range(1, NB):
            acc = acc + wc_ref[r, b] * w[b]
        wall_ref[r] = acc


def _wcomb(w_comp, weight):
    return pl.pallas_call(
        _wcomb_body,
        out_shape=jax.ShapeDtypeStruct((R, D, D), jnp.float32),
        in_specs=[
            pl.BlockSpec(memory_space=pltpu.SMEM),
            pl.BlockSpec(memory_space=pltpu.VMEM),
        ],
        out_specs=pl.BlockSpec(memory_space=pltpu.VMEM),
    )(w_comp, weight)


BN = 400


def _prep_body(x_ref, wall_ref, lw_ref, bias_ref, y2_ref, hloop_ref):
    xb = x_ref[...]
    for r in range(R):
        y2_ref[:, r, :] = jnp.dot(xb, wall_ref[r], preferred_element_type=jnp.float32)
    hloop_ref[...] = (
        jnp.dot(xb, lw_ref[...], preferred_element_type=jnp.float32) + bias_ref[...]
    )


def _prep(x, wall, loop_weight, h_bias2d):
    grid = N // BN
    return pl.pallas_call(
        _prep_body,
        grid=(grid,),
        in_specs=[
            pl.BlockSpec((BN, D), lambda i: (i, 0)),
            pl.BlockSpec((R, D, D), lambda i: (0, 0, 0)),
            pl.BlockSpec((D, D), lambda i: (0, 0)),
            pl.BlockSpec((1, D), lambda i: (0, 0)),
        ],
        out_specs=[
            pl.BlockSpec((BN, R, D), lambda i: (i, 0, 0)),
            pl.BlockSpec((BN, D), lambda i: (i, 0)),
        ],
        out_shape=[
            jax.ShapeDtypeStruct((N, R, D), jnp.float32),
            jax.ShapeDtypeStruct((N, D), jnp.float32),
        ],
    )(x, wall, loop_weight, h_bias2d)


def _gidx_body(src_ref, et_ref, gidx_ref):
    gidx_ref[...] = src_ref[...] * R + et_ref[...]


def _gidx(src2d, et2d):
    return pl.pallas_call(
        _gidx_body,
        out_shape=jax.ShapeDtypeStruct(src2d.shape, jnp.int32),
    )(src2d, et2d)


def _sc_body(y2_hbm, gidx_hbm, dst_hbm, zeros_hbm, out_hbm,
             gidx_v, dst_v, rows0, rows1, h_sh, sem0, sem1):
    cid = lax.axis_index("c")
    sid = lax.axis_index("s")
    wid = sid * NC + cid
    rbase = sid * ZB

    @pl.when(sid < NS - 1)
    def _():
        pltpu.sync_copy(zeros_hbm, h_sh.at[pl.ds(rbase, ZB)])

    @pl.when(sid == NS - 1)
    def _():
        pltpu.sync_copy(zeros_hbm.at[pl.ds(0, ZL)], h_sh.at[pl.ds(rbase, ZL)])

    plsc.subcore_barrier()

    bufs = ((rows0, sem0), (rows1, sem1))
    for h in range(PH):
        pltpu.sync_copy(gidx_hbm.at[wid, h], gidx_v)
        pltpu.sync_copy(dst_hbm.at[wid, h], dst_v)
        for b in range(2):
            pltpu.async_copy(y2_hbm.at[gidx_v.at[b]], bufs[b][0], bufs[b][1])

        def pair(i, carry):
            for b in range(2):
                j = 2 * i + b
                buf, sem = bufs[b]
                pltpu.make_async_copy(y2_hbm.at[gidx_v.at[j]], buf, sem).wait()
                pltpu.sync_copy(buf, h_sh.at[dst_v.at[j]], add=True)

                @pl.when(j + 2 < CPP)
                def _():
                    pltpu.async_copy(y2_hbm.at[gidx_v.at[j + 2]], buf, sem)

            return carry

        lax.fori_loop(0, CPP // 2, pair, 0)
    plsc.subcore_barrier()

    @pl.when(sid < NS - 1)
    def _():
        pltpu.sync_copy(h_sh.at[pl.ds(rbase, ZB)], out_hbm.at[cid].at[pl.ds(rbase, ZB)])

    @pl.when(sid == NS - 1)
    def _():
        pltpu.sync_copy(h_sh.at[pl.ds(rbase, ZL)], out_hbm.at[cid].at[pl.ds(rbase, ZL)])


@functools.cache
def _sc_scatter_kernel():
    return pl.kernel(
        _sc_body,
        out_type=jax.ShapeDtypeStruct((NC, N, D), jnp.float32),
        mesh=plsc.VectorSubcoreMesh(
            core_axis_name="c", subcore_axis_name="s", num_cores=NC, num_subcores=NS
        ),
        scratch_types=[
            pltpu.VMEM((CPP, K), jnp.int32),
            pltpu.VMEM((CPP, K), jnp.int32),
            pltpu.VMEM((K, D), jnp.float32),
            pltpu.VMEM((K, D), jnp.float32),
            pltpu.VMEM_SHARED((N, D), jnp.float32),
            pltpu.SemaphoreType.DMA,
            pltpu.SemaphoreType.DMA,
        ],
    )


FBN = 2000


def _final_body(p_ref, hl_ref, out_ref):
    out_ref[...] = p_ref[0] + p_ref[1] + hl_ref[...]


def _final(partial, h_loop):
    return pl.pallas_call(
        _final_body,
        grid=(N // FBN,),
        in_specs=[
            pl.BlockSpec((NC, FBN, D), lambda i: (0, i, 0)),
            pl.BlockSpec((FBN, D), lambda i: (i, 0)),
        ],
        out_specs=pl.BlockSpec((FBN, D), lambda i: (i, 0)),
        out_shape=jax.ShapeDtypeStruct((N, D), jnp.float32),
    )(partial, h_loop)


def kernel(x, edge_index, etypes, weight, w_comp, loop_weight, h_bias):
    wall = _wcomb(w_comp, weight)
    y2, h_loop = _prep(x, wall, loop_weight, h_bias.reshape(1, D))
    gidx = _gidx(edge_index[0].reshape(E // D, D), etypes.reshape(E // D, D))
    partial = _sc_scatter_kernel()(
        y2.reshape(N * R, D),
        gidx.reshape(NW, PH, CPP, K),
        edge_index[1].reshape(NW, PH, CPP, K),
        jnp.zeros((ZB, D), jnp.float32),
    )
    return _final(partial, h_loop)
